# Initial kernel scaffold; baseline (speedup 1.0000x reference)
import jax
import jax.numpy as jnp
from jax.experimental import pallas as pl


def kernel(xyz, features, w_s0_0, g_s0_0, b_s0_0, w_s0_1, g_s0_1, b_s0_1, w_s1_0, g_s1_0, b_s1_0, w_s1_1, g_s1_1, b_s1_1, w_agg, g_agg, b_agg):
    raise NotImplementedError("write your pallas kernel here")


# speedup vs baseline: 1.7247x; 1.7247x over previous
import functools

import jax
import jax.numpy as jnp
from jax.experimental import pallas as pl
from jax.experimental.pallas import tpu as pltpu

_B = 2
_N = 8192
_C_IN = 32
_NPOINT = 2048
_RADII = (0.5, 1.0)
_NSAMPLES = (16, 32)
_EPS = 1e-5

_SUB = 8
_LANE = _N // _SUB


def _fps_body(xq_ref, out_ref):
    flat = (jax.lax.broadcasted_iota(jnp.int32, (_SUB, _LANE), 0) * _LANE
            + jax.lax.broadcasted_iota(jnp.int32, (_SUB, _LANE), 1))
    slot = (jax.lax.broadcasted_iota(jnp.int32, (16, _NPOINT // 16), 0)
            * (_NPOINT // 16)
            + jax.lax.broadcasted_iota(jnp.int32, (16, _NPOINT // 16), 1))
    big = jnp.int32(_N)

    def one_batch(b):
        x = xq_ref[b, 0]
        y = xq_ref[b, 1]
        z = xq_ref[b, 2]

        def body(i, st):
            dist, far, nx, ny, nz = st
            sel = (flat == far).astype(jnp.float32)
            cx = jnp.sum(x * sel)
            cy = jnp.sum(y * sel)
            cz = jnp.sum(z * sel)
            here = slot == i
            nx = jnp.where(here, cx, nx)
            ny = jnp.where(here, cy, ny)
            nz = jnp.where(here, cz, nz)
            dx = x - cx
            dy = y - cy
            dz = z - cz
            d = dx * dx + dy * dy + dz * dz
            dist = jnp.minimum(dist, d)
            m = jnp.max(dist)
            far = jnp.min(jnp.where(dist == m, flat, big))
            return dist, far, nx, ny, nz

        dist0 = jnp.full((_SUB, _LANE), 1e10, jnp.float32)
        zeros = jnp.zeros((16, _NPOINT // 16), jnp.float32)
        _, _, nx, ny, nz = jax.lax.fori_loop(
            0, _NPOINT, body, (dist0, jnp.int32(0), zeros, zeros, zeros))
        out_ref[b, 0] = nx
        out_ref[b, 1] = ny
        out_ref[b, 2] = nz

    one_batch(0)
    one_batch(1)


def _fps(xyz):
    xq = jnp.transpose(xyz, (0, 2, 1)).reshape(_B, 3, _SUB, _LANE)
    out = pl.pallas_call(
        _fps_body,
        out_shape=jax.ShapeDtypeStruct((_B, 3, 16, _NPOINT // 16), jnp.float32),
        in_specs=[pl.BlockSpec(memory_space=pltpu.VMEM)],
        out_specs=pl.BlockSpec(memory_space=pltpu.VMEM),
    )(xq)
    return jnp.transpose(out.reshape(_B, 3, _NPOINT), (0, 2, 1))


def _ball_group(xyz, new_xyz, features, radius, nsample):
    b, n, _ = xyz.shape
    d2 = jnp.sum((new_xyz[:, :, None, :] - xyz[:, None, :, :]) ** 2, axis=-1)
    mask = d2 < radius * radius
    ar = jnp.arange(n, dtype=jnp.int32)
    vals = jnp.where(mask, ar[None, None, :], n)
    idx_sorted = jnp.sort(vals, axis=-1)[:, :, :nsample]
    idx_cnt = jnp.minimum(jnp.sum(mask, axis=-1), nsample)
    first = idx_sorted[:, :, :1]
    idx = jnp.where(idx_sorted >= n, jnp.broadcast_to(first, idx_sorted.shape), idx_sorted)
    idx = jnp.where(idx >= n, 0, idx).astype(jnp.int32)
    bids = jnp.arange(b)[:, None, None]
    grouped_xyz = xyz[bids, idx] - new_xyz[:, :, None, :]
    feat_t = jnp.transpose(features, (0, 2, 1))
    grouped_feat = jnp.transpose(feat_t[bids, idx], (0, 3, 1, 2))
    gx = jnp.transpose(grouped_xyz, (0, 3, 1, 2))
    return idx_cnt, jnp.concatenate([gx, grouped_feat], axis=1)


def _conv_bn_relu_2d(x, w, g, b):
    y = jnp.einsum('bcsn,oc->bosn', x, w)
    mean = jnp.mean(y, axis=(0, 2, 3), keepdims=True)
    var = jnp.var(y, axis=(0, 2, 3), keepdims=True)
    y = (y - mean) / jnp.sqrt(var + _EPS) * g[None, :, None, None] + b[None, :, None, None]
    return jnp.maximum(y, 0.0)


def kernel(xyz, features, w_s0_0, g_s0_0, b_s0_0, w_s0_1, g_s0_1, b_s0_1,
           w_s1_0, g_s1_0, b_s1_0, w_s1_1, g_s1_1, b_s1_1, w_agg, g_agg, b_agg):
    new_xyz = _fps(xyz)
    params = [[(w_s0_0, g_s0_0, b_s0_0), (w_s0_1, g_s0_1, b_s0_1)],
              [(w_s1_0, g_s1_0, b_s1_0), (w_s1_1, g_s1_1, b_s1_1)]]
    outs = []
    for i in range(2):
        idx_cnt, nf = _ball_group(xyz, new_xyz, features, _RADII[i], _NSAMPLES[i])
        for (w, g, b) in params[i]:
            nf = _conv_bn_relu_2d(nf, w, g, b)
        maskc = (idx_cnt > 0).astype(nf.dtype)[:, None, :, None]
        nf = nf * maskc
        outs.append(jnp.max(nf, axis=3))
    nf = jnp.concatenate(outs, axis=1)
    y = jnp.einsum('bcs,oc->bos', nf, w_agg)
    mean = jnp.mean(y, axis=(0, 2), keepdims=True)
    var = jnp.var(y, axis=(0, 2), keepdims=True)
    y = (y - mean) / jnp.sqrt(var + _EPS) * g_agg[None, :, None] + b_agg[None, :, None]
    y = jnp.maximum(y, 0.0)
    return new_xyz, y


# speedup vs baseline: 5.0198x; 2.9104x over previous
import functools

import jax
import jax.numpy as jnp
from jax.experimental import pallas as pl
from jax.experimental.pallas import tpu as pltpu

_B = 2
_N = 8192
_C_IN = 32
_NPOINT = 2048
_RADII = (0.5, 1.0)
_NSAMPLES = (16, 32)
_EPS = 1e-5

_SUB = 8
_LANE = _N // _SUB


def _fps_body(xq_ref, out_ref):
    flat = (jax.lax.broadcasted_iota(jnp.int32, (_SUB, _LANE), 0) * _LANE
            + jax.lax.broadcasted_iota(jnp.int32, (_SUB, _LANE), 1))
    slot = (jax.lax.broadcasted_iota(jnp.int32, (16, _NPOINT // 16), 0)
            * (_NPOINT // 16)
            + jax.lax.broadcasted_iota(jnp.int32, (16, _NPOINT // 16), 1))
    big = jnp.int32(_N)

    def one_batch(b):
        x = xq_ref[b, 0]
        y = xq_ref[b, 1]
        z = xq_ref[b, 2]

        def body(i, st):
            dist, far, nx, ny, nz = st
            sel = (flat == far).astype(jnp.float32)
            cx = jnp.sum(x * sel)
            cy = jnp.sum(y * sel)
            cz = jnp.sum(z * sel)
            here = slot == i
            nx = jnp.where(here, cx, nx)
            ny = jnp.where(here, cy, ny)
            nz = jnp.where(here, cz, nz)
            dx = x - cx
            dy = y - cy
            dz = z - cz
            d = dx * dx + dy * dy + dz * dz
            dist = jnp.minimum(dist, d)
            m = jnp.max(dist)
            far = jnp.min(jnp.where(dist == m, flat, big))
            return dist, far, nx, ny, nz

        dist0 = jnp.full((_SUB, _LANE), 1e10, jnp.float32)
        zeros = jnp.zeros((16, _NPOINT // 16), jnp.float32)
        _, _, nx, ny, nz = jax.lax.fori_loop(
            0, _NPOINT, body, (dist0, jnp.int32(0), zeros, zeros, zeros))
        out_ref[b, 0] = nx
        out_ref[b, 1] = ny
        out_ref[b, 2] = nz

    one_batch(0)
    one_batch(1)


def _fps(xq):
    out = pl.pallas_call(
        _fps_body,
        out_shape=jax.ShapeDtypeStruct((_B, 3, 16, _NPOINT // 16), jnp.float32),
        in_specs=[pl.BlockSpec(memory_space=pltpu.VMEM)],
        out_specs=pl.BlockSpec(memory_space=pltpu.VMEM),
    )(xq)
    return jnp.transpose(out.reshape(_B, 3, _NPOINT), (0, 2, 1))


_BQ_CB = 256
_BQ_CHUNK = 1024
_BIG = 1 << 30


def _bq_body(xq_ref, nxyz_ref, idx0_ref, idx1_ref, v0_ref, v1_ref, d2_ref):
    CB = _BQ_CB
    lane = jax.lax.broadcasted_iota(jnp.int32, (1, _BQ_CHUNK), 1)
    col16 = jax.lax.broadcasted_iota(jnp.int32, (CB, 16), 1)
    col32 = jax.lax.broadcasted_iota(jnp.int32, (CB, 32), 1)
    nchunks = _N // _BQ_CHUNK

    def wcond(st):
        k, _, cnt0, _, _, cnt1, _ = st
        todo = (jnp.min(cnt0) < 16) | (jnp.min(cnt1) < 32)
        return (k < nchunks) & todo

    def wbody(st):
        k, cur0, cnt0, reg0, cur1, cnt1, reg1 = st
        base = k * _BQ_CHUNK
        acc = jnp.zeros((CB, _BQ_CHUNK), jnp.float32)
        for c in range(3):
            row = xq_ref[0, c, pl.ds(k, 1), :]
            colv = nxyz_ref[0, c, :, :]
            dfc = colv - row
            acc = acc + dfc * dfc
        d2_ref[:, :] = acc
        pidx = lane + base

        def mk(ns, r2, col_iota):
            def sbody(_, sst):
                cur, cnt, reg = sst
                d2v = d2_ref[:, :]
                okm = (d2v < r2) & (pidx > cur)
                cand = jnp.where(okm, pidx, _BIG)
                nxt = jnp.min(cand, axis=1, keepdims=True)
                found = (nxt < _BIG) & (cnt < ns)
                reg = jnp.where((col_iota == cnt) & found, nxt, reg)
                cnt = cnt + found.astype(jnp.int32)
                cur = jnp.where(found, nxt, cur)
                return cur, cnt, reg
            return sbody

        cur0, cnt0, reg0 = jax.lax.fori_loop(
            0, 16, mk(16, jnp.float32(_RADII[0] * _RADII[0]), col16),
            (cur0, cnt0, reg0))
        cur1, cnt1, reg1 = jax.lax.fori_loop(
            0, 32, mk(32, jnp.float32(_RADII[1] * _RADII[1]), col32),
            (cur1, cnt1, reg1))
        return k + 1, cur0, cnt0, reg0, cur1, cnt1, reg1

    colz = jnp.full((CB, 1), -1, jnp.int32)
    cntz = jnp.zeros((CB, 1), jnp.int32)
    st = (jnp.int32(0), colz, cntz, jnp.full((CB, 16), _BIG, jnp.int32),
          colz, cntz, jnp.full((CB, 32), _BIG, jnp.int32))
    _, _, cnt0, reg0, _, cnt1, reg1 = jax.lax.while_loop(wcond, wbody, st)

    pad0 = jnp.where(cnt0 > 0, reg0[:, 0:1], 0)
    idx0_ref[0] = jnp.where(col16 < cnt0, reg0, pad0)
    v0_ref[0] = (cnt0 > 0).astype(jnp.float32)
    pad1 = jnp.where(cnt1 > 0, reg1[:, 0:1], 0)
    idx1_ref[0] = jnp.where(col32 < cnt1, reg1, pad1)
    v1_ref[0] = (cnt1 > 0).astype(jnp.float32)


def _ball_query(xq, new_xyz):
    nxyz4 = jnp.transpose(new_xyz, (0, 2, 1))[..., None]
    nblk = _NPOINT // _BQ_CB
    idx0, idx1, v0, v1 = pl.pallas_call(
        _bq_body,
        grid=(_B, nblk),
        in_specs=[
            pl.BlockSpec((1, 3, _SUB, _LANE), lambda b, j: (b, 0, 0, 0)),
            pl.BlockSpec((1, 3, _BQ_CB, 1), lambda b, j: (b, 0, j, 0)),
        ],
        out_specs=[
            pl.BlockSpec((1, _BQ_CB, 16), lambda b, j: (b, j, 0)),
            pl.BlockSpec((1, _BQ_CB, 32), lambda b, j: (b, j, 0)),
            pl.BlockSpec((1, _BQ_CB, 1), lambda b, j: (b, j, 0)),
            pl.BlockSpec((1, _BQ_CB, 1), lambda b, j: (b, j, 0)),
        ],
        out_shape=[
            jax.ShapeDtypeStruct((_B, _NPOINT, 16), jnp.int32),
            jax.ShapeDtypeStruct((_B, _NPOINT, 32), jnp.int32),
            jax.ShapeDtypeStruct((_B, _NPOINT, 1), jnp.float32),
            jax.ShapeDtypeStruct((_B, _NPOINT, 1), jnp.float32),
        ],
        scratch_shapes=[pltpu.VMEM((_BQ_CB, _BQ_CHUNK), jnp.float32)],
    )(xq, nxyz4)
    return idx0, idx1, v0[..., 0], v1[..., 0]


def _group_with_idx(xyz, new_xyz, features, idx):
    b = xyz.shape[0]
    bids = jnp.arange(b)[:, None, None]
    grouped_xyz = xyz[bids, idx] - new_xyz[:, :, None, :]
    feat_t = jnp.transpose(features, (0, 2, 1))
    grouped_feat = jnp.transpose(feat_t[bids, idx], (0, 3, 1, 2))
    gx = jnp.transpose(grouped_xyz, (0, 3, 1, 2))
    return jnp.concatenate([gx, grouped_feat], axis=1)


def _conv_bn_relu_2d(x, w, g, b):
    y = jnp.einsum('bcsn,oc->bosn', x, w)
    mean = jnp.mean(y, axis=(0, 2, 3), keepdims=True)
    var = jnp.var(y, axis=(0, 2, 3), keepdims=True)
    y = (y - mean) / jnp.sqrt(var + _EPS) * g[None, :, None, None] + b[None, :, None, None]
    return jnp.maximum(y, 0.0)


def kernel(xyz, features, w_s0_0, g_s0_0, b_s0_0, w_s0_1, g_s0_1, b_s0_1,
           w_s1_0, g_s1_0, b_s1_0, w_s1_1, g_s1_1, b_s1_1, w_agg, g_agg, b_agg):
    xq = jnp.transpose(xyz, (0, 2, 1)).reshape(_B, 3, _SUB, _LANE)
    new_xyz = _fps(xq)
    idx0, idx1, v0, v1 = _ball_query(xq, new_xyz)
    idxs, valids = (idx0, idx1), (v0, v1)
    params = [[(w_s0_0, g_s0_0, b_s0_0), (w_s0_1, g_s0_1, b_s0_1)],
              [(w_s1_0, g_s1_0, b_s1_0), (w_s1_1, g_s1_1, b_s1_1)]]
    outs = []
    for i in range(2):
        nf = _group_with_idx(xyz, new_xyz, features, idxs[i])
        for (w, g, b) in params[i]:
            nf = _conv_bn_relu_2d(nf, w, g, b)
        maskc = valids[i][:, None, :, None]
        nf = nf * maskc
        outs.append(jnp.max(nf, axis=3))
    nf = jnp.concatenate(outs, axis=1)
    y = jnp.einsum('bcs,oc->bos', nf, w_agg)
    mean = jnp.mean(y, axis=(0, 2), keepdims=True)
    var = jnp.var(y, axis=(0, 2), keepdims=True)
    y = (y - mean) / jnp.sqrt(var + _EPS) * g_agg[None, :, None] + b_agg[None, :, None]
    y = jnp.maximum(y, 0.0)
    return new_xyz, y


# speedup vs baseline: 14.8372x; 2.9557x over previous
import functools

import jax
import jax.numpy as jnp
from jax.experimental import pallas as pl
from jax.experimental.pallas import tpu as pltpu
from jax.experimental.pallas import tpu_sc as plsc

_B = 2
_N = 8192
_C_IN = 32
_NPOINT = 2048
_RADII = (0.5, 1.0)
_NSAMPLES = (16, 32)
_EPS = 1e-5

_SUB = 8
_LANE = _N // _SUB


def _fps_body(xq_ref, out_ref):
    flat = (jax.lax.broadcasted_iota(jnp.int32, (_SUB, _LANE), 0) * _LANE
            + jax.lax.broadcasted_iota(jnp.int32, (_SUB, _LANE), 1))
    slot = (jax.lax.broadcasted_iota(jnp.int32, (16, _NPOINT // 16), 0)
            * (_NPOINT // 16)
            + jax.lax.broadcasted_iota(jnp.int32, (16, _NPOINT // 16), 1))
    big = jnp.int32(_N)

    def one_batch(b):
        x = xq_ref[b, 0]
        y = xq_ref[b, 1]
        z = xq_ref[b, 2]

        def body(i, st):
            dist, far, nx, ny, nz = st
            sel = (flat == far).astype(jnp.float32)
            cx = jnp.sum(x * sel)
            cy = jnp.sum(y * sel)
            cz = jnp.sum(z * sel)
            here = slot == i
            nx = jnp.where(here, cx, nx)
            ny = jnp.where(here, cy, ny)
            nz = jnp.where(here, cz, nz)
            dx = x - cx
            dy = y - cy
            dz = z - cz
            d = dx * dx + dy * dy + dz * dz
            dist = jnp.minimum(dist, d)
            m = jnp.max(dist)
            far = jnp.min(jnp.where(dist == m, flat, big))
            return dist, far, nx, ny, nz

        dist0 = jnp.full((_SUB, _LANE), 1e10, jnp.float32)
        zeros = jnp.zeros((16, _NPOINT // 16), jnp.float32)
        _, _, nx, ny, nz = jax.lax.fori_loop(
            0, _NPOINT, body, (dist0, jnp.int32(0), zeros, zeros, zeros))
        out_ref[b, 0] = nx
        out_ref[b, 1] = ny
        out_ref[b, 2] = nz

    one_batch(0)
    one_batch(1)


def _fps(xq):
    out = pl.pallas_call(
        _fps_body,
        out_shape=jax.ShapeDtypeStruct((_B, 3, 16, _NPOINT // 16), jnp.float32),
        in_specs=[pl.BlockSpec(memory_space=pltpu.VMEM)],
        out_specs=pl.BlockSpec(memory_space=pltpu.VMEM),
    )(xq)
    return jnp.transpose(out.reshape(_B, 3, _NPOINT), (0, 2, 1))


_BQ_CB = 256
_BQ_CHUNK = 1024
_BIG = 1 << 30


def _bq_body(xq_ref, nxyz_ref, idx0_ref, idx1_ref, v0_ref, v1_ref, d2_ref):
    CB = _BQ_CB
    lane = jax.lax.broadcasted_iota(jnp.int32, (1, _BQ_CHUNK), 1)
    col16 = jax.lax.broadcasted_iota(jnp.int32, (CB, 16), 1)
    col32 = jax.lax.broadcasted_iota(jnp.int32, (CB, 32), 1)
    nchunks = _N // _BQ_CHUNK

    def wcond(st):
        k, _, cnt0, _, _, cnt1, _ = st
        todo = (jnp.min(cnt0) < 16) | (jnp.min(cnt1) < 32)
        return (k < nchunks) & todo

    def wbody(st):
        k, cur0, cnt0, reg0, cur1, cnt1, reg1 = st
        base = k * _BQ_CHUNK
        acc = jnp.zeros((CB, _BQ_CHUNK), jnp.float32)
        for c in range(3):
            row = xq_ref[0, c, pl.ds(k, 1), :]
            colv = nxyz_ref[0, c, :, :]
            dfc = colv - row
            acc = acc + dfc * dfc
        d2_ref[:, :] = acc
        pidx = lane + base

        def mk(ns, r2, col_iota):
            def sbody(_, sst):
                cur, cnt, reg = sst
                d2v = d2_ref[:, :]
                okm = (d2v < r2) & (pidx > cur)
                cand = jnp.where(okm, pidx, _BIG)
                nxt = jnp.min(cand, axis=1, keepdims=True)
                found = (nxt < _BIG) & (cnt < ns)
                reg = jnp.where((col_iota == cnt) & found, nxt, reg)
                cnt = cnt + found.astype(jnp.int32)
                cur = jnp.where(found, nxt, cur)
                return cur, cnt, reg
            return sbody

        cur0, cnt0, reg0 = jax.lax.fori_loop(
            0, 16, mk(16, jnp.float32(_RADII[0] * _RADII[0]), col16),
            (cur0, cnt0, reg0))
        cur1, cnt1, reg1 = jax.lax.fori_loop(
            0, 32, mk(32, jnp.float32(_RADII[1] * _RADII[1]), col32),
            (cur1, cnt1, reg1))
        return k + 1, cur0, cnt0, reg0, cur1, cnt1, reg1

    colz = jnp.full((CB, 1), -1, jnp.int32)
    cntz = jnp.zeros((CB, 1), jnp.int32)
    st = (jnp.int32(0), colz, cntz, jnp.full((CB, 16), _BIG, jnp.int32),
          colz, cntz, jnp.full((CB, 32), _BIG, jnp.int32))
    _, _, cnt0, reg0, _, cnt1, reg1 = jax.lax.while_loop(wcond, wbody, st)

    boff = pl.program_id(0) * _N
    pad0 = jnp.where(cnt0 > 0, reg0[:, 0:1], 0)
    idx0_ref[0] = jnp.where(col16 < cnt0, reg0, pad0) + boff
    v0_ref[0] = (cnt0 > 0).astype(jnp.float32)
    pad1 = jnp.where(cnt1 > 0, reg1[:, 0:1], 0)
    idx1_ref[0] = jnp.where(col32 < cnt1, reg1, pad1) + boff
    v1_ref[0] = (cnt1 > 0).astype(jnp.float32)


def _ball_query(xq, new_xyz):
    nxyz4 = jnp.transpose(new_xyz, (0, 2, 1))[..., None]
    nblk = _NPOINT // _BQ_CB
    idx0, idx1, v0, v1 = pl.pallas_call(
        _bq_body,
        grid=(_B, nblk),
        in_specs=[
            pl.BlockSpec((1, 3, _SUB, _LANE), lambda b, j: (b, 0, 0, 0)),
            pl.BlockSpec((1, 3, _BQ_CB, 1), lambda b, j: (b, 0, j, 0)),
        ],
        out_specs=[
            pl.BlockSpec((1, _BQ_CB, 16), lambda b, j: (b, j, 0)),
            pl.BlockSpec((1, _BQ_CB, 32), lambda b, j: (b, j, 0)),
            pl.BlockSpec((1, _BQ_CB, 1), lambda b, j: (b, j, 0)),
            pl.BlockSpec((1, _BQ_CB, 1), lambda b, j: (b, j, 0)),
        ],
        out_shape=[
            jax.ShapeDtypeStruct((_B, _NPOINT, 16), jnp.int32),
            jax.ShapeDtypeStruct((_B, _NPOINT, 32), jnp.int32),
            jax.ShapeDtypeStruct((_B, _NPOINT, 1), jnp.float32),
            jax.ShapeDtypeStruct((_B, _NPOINT, 1), jnp.float32),
        ],
        scratch_shapes=[pltpu.VMEM((_BQ_CB, _BQ_CHUNK), jnp.float32)],
    )(xq, nxyz4)
    return idx0, idx1, v0[..., 0], v1[..., 0]


_NC = 2
_NS = 16
_NW = _NC * _NS


def _sc_gather(table, idxw, d):
    k_per_w = idxw.shape[1]
    per_w = k_per_w * 128
    mesh = plsc.VectorSubcoreMesh(core_axis_name="c", subcore_axis_name="s",
                                  num_cores=_NC, num_subcores=_NS)

    @functools.partial(
        pl.kernel, mesh=mesh,
        out_type=jax.ShapeDtypeStruct((_NW * per_w, d), jnp.float32),
        scratch_types=[
            pltpu.VMEM((k_per_w, 128), jnp.int32),
            pltpu.VMEM((128, d), jnp.float32),
            pltpu.VMEM((128, d), jnp.float32),
            pltpu.SemaphoreType.DMA,
            pltpu.SemaphoreType.DMA,
        ],
    )
    def gk(idx_hbm, table_hbm, out_hbm, idx_v, buf0, buf1, sem0, sem1):
        cid = jax.lax.axis_index("c")
        sid = jax.lax.axis_index("s")
        wid = sid * _NC + cid
        base = wid * per_w
        pltpu.sync_copy(idx_hbm.at[wid], idx_v)
        bufs = (buf0, buf1)
        sems = (sem0, sem1)
        handles = [None, None]
        handles[0] = pltpu.async_copy(table_hbm.at[idx_v.at[0]], buf0, sem0)
        for j in range(k_per_w):
            nx = j + 1
            if nx < k_per_w:
                handles[nx % 2] = pltpu.async_copy(
                    table_hbm.at[idx_v.at[nx]], bufs[nx % 2], sems[nx % 2])
            handles[j % 2].wait()
            pltpu.sync_copy(bufs[j % 2], out_hbm.at[pl.ds(base + j * 128, 128)])

    return gk(idxw, table)


_CBM = 128


def _mk_z0_body(ns):
    def body(g_ref, nx_ref, w_ref, z_ref, st_ref):
        j = pl.program_id(0)
        g = g_ref[...]
        dx = g[:, :, 0:3] - nx_ref[...][:, None, :]
        x = jnp.concatenate([dx, g[:, :, 3:35]], axis=2)
        xb = x.reshape(_CBM * ns, 35).astype(jnp.bfloat16)
        z = jnp.dot(xb, w_ref[...].astype(jnp.bfloat16),
                    preferred_element_type=jnp.float32)
        z = z.reshape(_CBM, ns, -1)
        z_ref[...] = z

        @pl.when(j == 0)
        def _():
            st_ref[...] = jnp.zeros_like(st_ref)

        st_ref[0:1, :] += jnp.sum(z, axis=(0, 1))[None]
        st_ref[1:2, :] += jnp.sum(z * z, axis=(0, 1))[None]
    return body


def _layer0(g, nxf, wt, ns, o):
    grid = (_B * _NPOINT) // _CBM
    return pl.pallas_call(
        _mk_z0_body(ns),
        grid=(grid,),
        in_specs=[
            pl.BlockSpec((_CBM, ns, 128), lambda j: (j, 0, 0)),
            pl.BlockSpec((_CBM, 3), lambda j: (j, 0)),
            pl.BlockSpec((35, o), lambda j: (0, 0)),
        ],
        out_specs=[
            pl.BlockSpec((_CBM, ns, o), lambda j: (j, 0, 0)),
            pl.BlockSpec((2, o), lambda j: (0, 0)),
        ],
        out_shape=[
            jax.ShapeDtypeStruct((_B * _NPOINT, ns, o), jnp.float32),
            jax.ShapeDtypeStruct((2, o), jnp.float32),
        ],
    )(g, nxf, wt)


def _mk_l1_body(ns, o, o1):
    def body(z_ref, ab_ref, w1_ref, z1_ref, st_ref):
        j = pl.program_id(0)
        a = ab_ref[0:1, :]
        b = ab_ref[1:2, :]
        h = jnp.maximum(z_ref[...] * a[None] + b[None], 0.0)
        z1 = jnp.dot(h.reshape(_CBM * ns, o).astype(jnp.bfloat16),
                     w1_ref[...].astype(jnp.bfloat16),
                     preferred_element_type=jnp.float32)
        z1 = z1.reshape(_CBM, ns, o1)
        z1_ref[...] = z1

        @pl.when(j == 0)
        def _():
            st_ref[...] = jnp.zeros_like(st_ref)

        st_ref[0:1, :] += jnp.sum(z1, axis=(0, 1))[None]
        st_ref[1:2, :] += jnp.sum(z1 * z1, axis=(0, 1))[None]
    return body


def _layer1(z, ab, w1t, ns, o, o1):
    grid = (_B * _NPOINT) // _CBM
    return pl.pallas_call(
        _mk_l1_body(ns, o, o1),
        grid=(grid,),
        in_specs=[
            pl.BlockSpec((_CBM, ns, o), lambda j: (j, 0, 0)),
            pl.BlockSpec((2, o), lambda j: (0, 0)),
            pl.BlockSpec((o, o1), lambda j: (0, 0)),
        ],
        out_specs=[
            pl.BlockSpec((_CBM, ns, o1), lambda j: (j, 0, 0)),
            pl.BlockSpec((2, o1), lambda j: (0, 0)),
        ],
        out_shape=[
            jax.ShapeDtypeStruct((_B * _NPOINT, ns, o1), jnp.float32),
            jax.ShapeDtypeStruct((2, o1), jnp.float32),
        ],
    )(z, ab, w1t)


def _pool_body(z_ref, ab_ref, v_ref, out_ref):
    a = ab_ref[0:1, :]
    b = ab_ref[1:2, :]
    h = jnp.maximum(z_ref[...] * a[None] + b[None], 0.0)
    h = h * v_ref[...][:, :, None]
    out_ref[...] = jnp.max(h, axis=1)


def _pool(z1, ab, valid, ns, o1):
    grid = (_B * _NPOINT) // _CBM
    return pl.pallas_call(
        _pool_body,
        grid=(grid,),
        in_specs=[
            pl.BlockSpec((_CBM, ns, o1), lambda j: (j, 0, 0)),
            pl.BlockSpec((2, o1), lambda j: (0, 0)),
            pl.BlockSpec((_CBM, 1), lambda j: (j, 0)),
        ],
        out_specs=pl.BlockSpec((_CBM, o1), lambda j: (j, 0)),
        out_shape=jax.ShapeDtypeStruct((_B * _NPOINT, o1), jnp.float32),
    )(z1, ab, valid)


def _agg_body(x_ref, w_ref, z_ref, st_ref):
    j = pl.program_id(0)
    z = jnp.dot(x_ref[...].astype(jnp.bfloat16),
                w_ref[...].astype(jnp.bfloat16),
                preferred_element_type=jnp.float32)
    z_ref[...] = z

    @pl.when(j == 0)
    def _():
        st_ref[...] = jnp.zeros_like(st_ref)

    st_ref[0:1, :] += jnp.sum(z, axis=0)[None]
    st_ref[1:2, :] += jnp.sum(z * z, axis=0)[None]


def _agg(x, wt):
    grid = (_B * _NPOINT) // 512
    return pl.pallas_call(
        _agg_body,
        grid=(grid,),
        in_specs=[
            pl.BlockSpec((512, 128), lambda j: (j, 0)),
            pl.BlockSpec((128, 128), lambda j: (0, 0)),
        ],
        out_specs=[
            pl.BlockSpec((512, 128), lambda j: (j, 0)),
            pl.BlockSpec((2, 128), lambda j: (0, 0)),
        ],
        out_shape=[
            jax.ShapeDtypeStruct((_B * _NPOINT, 128), jnp.float32),
            jax.ShapeDtypeStruct((2, 128), jnp.float32),
        ],
    )(x, wt)


def _fin_body(z_ref, ab_ref, y_ref):
    a = ab_ref[0:1, :]
    b = ab_ref[1:2, :]
    y_ref[...] = jnp.maximum(z_ref[...] * a + b, 0.0)


def _finalize(z, ab):
    grid = (_B * _NPOINT) // 512
    return pl.pallas_call(
        _fin_body,
        grid=(grid,),
        in_specs=[
            pl.BlockSpec((512, 128), lambda j: (j, 0)),
            pl.BlockSpec((2, 128), lambda j: (0, 0)),
        ],
        out_specs=pl.BlockSpec((512, 128), lambda j: (j, 0)),
        out_shape=jax.ShapeDtypeStruct((_B * _NPOINT, 128), jnp.float32),
    )(z, ab)


def _moments_to_affine(st, g, b, count):
    mean = st[0] / count
    var = jnp.maximum(st[1] / count - mean * mean, 0.0)
    a = g / jnp.sqrt(var + _EPS)
    return jnp.stack([a, b - mean * a])


def kernel(xyz, features, w_s0_0, g_s0_0, b_s0_0, w_s0_1, g_s0_1, b_s0_1,
           w_s1_0, g_s1_0, b_s1_0, w_s1_1, g_s1_1, b_s1_1, w_agg, g_agg, b_agg):
    xq = jnp.transpose(xyz, (0, 2, 1)).reshape(_B, 3, _SUB, _LANE)
    new_xyz = _fps(xq)
    idx0, idx1, v0, v1 = _ball_query(xq, new_xyz)

    feat_t = jnp.transpose(features, (0, 2, 1))
    a_all = jnp.concatenate([xyz, feat_t], axis=-1).reshape(_B * _N, 35)
    tab = jnp.pad(a_all, ((0, 0), (0, 93)))

    idxw0 = idx0.reshape(_NW, -1, 128)
    idxw1 = idx1.reshape(_NW, -1, 128)
    g0 = _sc_gather(tab, idxw0, 128).reshape(_B * _NPOINT, 16, 128)
    g1 = _sc_gather(tab, idxw1, 128).reshape(_B * _NPOINT, 32, 128)

    nxf = new_xyz.reshape(_B * _NPOINT, 3)
    vf0 = v0.reshape(_B * _NPOINT, 1)
    vf1 = v1.reshape(_B * _NPOINT, 1)

    m0 = float(_B * _NPOINT * 16)
    m1 = float(_B * _NPOINT * 32)

    z0, st = _layer0(g0, nxf, w_s0_0.T, 16, 32)
    ab = _moments_to_affine(st, g_s0_0, b_s0_0, m0)
    z0b, st = _layer1(z0, ab, w_s0_1.T, 16, 32, 64)
    ab = _moments_to_affine(st, g_s0_1, b_s0_1, m0)
    pooled0 = _pool(z0b, ab, vf0, 16, 64)

    z1, st = _layer0(g1, nxf, w_s1_0.T, 32, 64)
    ab = _moments_to_affine(st, g_s1_0, b_s1_0, m1)
    z1b, st = _layer1(z1, ab, w_s1_1.T, 32, 64, 64)
    ab = _moments_to_affine(st, g_s1_1, b_s1_1, m1)
    pooled1 = _pool(z1b, ab, vf1, 32, 64)

    x = jnp.concatenate([pooled0, pooled1], axis=1)
    z2, st = _agg(x, w_agg.T)
    ab = _moments_to_affine(st, g_agg, b_agg, float(_B * _NPOINT))
    y = _finalize(z2, ab)
    y = jnp.transpose(y.reshape(_B, _NPOINT, 128), (0, 2, 1))
    return new_xyz, y
